# Initial kernel scaffold; baseline (speedup 1.0000x reference)
import jax
import jax.numpy as jnp
from jax.experimental import pallas as pl


def kernel(charge, spin, dataset, charge_table, spin_table, dataset_table, bias):
    raise NotImplementedError("write your pallas kernel here")


# speedup vs baseline: 3.1693x; 3.1693x over previous
import functools

import jax
import jax.numpy as jnp
from jax import lax
from jax.experimental import pallas as pl
from jax.experimental.pallas import tpu as pltpu
from jax.experimental.pallas import tpu_sc as plsc

_B = 16384
_D = 128
_C = 128
_NC = 2
_NS = 16
_NW = _NC * _NS
_RPW = _B // _NW
_K = _RPW // _C


def _embed_body(charge_hbm, spin_hbm, dataset_hbm, ct_hbm, st_hbm, dt_hbm,
                bias_hbm, out_hbm, idx_c, idx_s, idx_d, rows_c, rows_s,
                rows_d, out_v, bias_v, sem):
    wid = lax.axis_index("s") * _NC + lax.axis_index("c")
    irow0 = wid * _K
    base = wid * _RPW

    pltpu.sync_copy(bias_hbm, bias_v)
    pltpu.sync_copy(charge_hbm.at[pl.ds(irow0, _K)], idx_c)
    pltpu.sync_copy(spin_hbm.at[pl.ds(irow0, _K)], idx_s)
    pltpu.sync_copy(dataset_hbm.at[pl.ds(irow0, _K)], idx_d)

    bias_regs = [bias_v[pl.ds(j * 16, 16)] for j in range(8)]

    for k in range(_K):
        cp_c = pltpu.async_copy(ct_hbm.at[idx_c.at[k]], rows_c, sem)
        cp_s = pltpu.async_copy(st_hbm.at[idx_s.at[k]], rows_s, sem)
        cp_d = pltpu.async_copy(dt_hbm.at[idx_d.at[k]], rows_d, sem)
        cp_c.wait()
        cp_s.wait()
        cp_d.wait()

        def row_body(r, carry):
            for j in range(8):
                sl = pl.ds(j * 16, 16)
                x = rows_c[r, sl] + rows_s[r, sl] + rows_d[r, sl] + bias_regs[j]
                out_v[r, sl] = x / (1.0 + jnp.exp(-x))
            return carry

        lax.fori_loop(0, _C, row_body, 0)

        pltpu.sync_copy(out_v, out_hbm.at[pl.ds(base + k * _C, _C)])


@jax.jit
def _embed(charge_idx, spin_idx, dataset_idx, charge_table, spin_table,
           dataset_table, bias):
    mesh = plsc.VectorSubcoreMesh(core_axis_name="c", subcore_axis_name="s")
    kern = pl.kernel(
        _embed_body,
        mesh=mesh,
        out_type=jax.ShapeDtypeStruct((_B, _D), jnp.float32),
        scratch_types=[
            pltpu.VMEM((_K, _C), jnp.int32),
            pltpu.VMEM((_K, _C), jnp.int32),
            pltpu.VMEM((_K, _C), jnp.int32),
            pltpu.VMEM((_C, _D), jnp.float32),
            pltpu.VMEM((_C, _D), jnp.float32),
            pltpu.VMEM((_C, _D), jnp.float32),
            pltpu.VMEM((_C, _D), jnp.float32),
            pltpu.VMEM((_D,), jnp.float32),
            pltpu.SemaphoreType.DMA,
        ],
    )
    return kern(charge_idx, spin_idx, dataset_idx, charge_table, spin_table,
                dataset_table, bias)


def kernel(charge, spin, dataset, charge_table, spin_table, dataset_table, bias):
    charge_idx = (charge + 100).reshape(_B // _C, _C)
    spin_idx = spin.reshape(_B // _C, _C)
    dataset_idx = dataset.reshape(_B // _C, _C)
    return _embed(charge_idx, spin_idx, dataset_idx, charge_table, spin_table,
                  dataset_table, bias)
